# baseline (speedup 1.0000x reference)
import functools

import jax
import jax.numpy as jnp
from jax import lax
from jax.experimental import pallas as pl
from jax.experimental.pallas import tpu as pltpu
from jax.experimental.pallas import tpu_sc as plsc

_NC = 2
_NS = 16
_L = 128


def _sc_gather(tab, idx2d):
    L = _L
    nidx = idx2d.shape[0] // L
    D = tab.shape[1]
    nw = _NC * _NS
    per_w = nidx // nw
    assert per_w * nw == nidx
    JC = 8
    nchunks = per_w // JC
    assert nchunks * JC == per_w and nchunks % 2 == 0
    crows = JC * L

    mesh = plsc.VectorSubcoreMesh(core_axis_name="c", subcore_axis_name="s")

    @functools.partial(
        pl.kernel,
        out_type=jax.ShapeDtypeStruct((nidx * L, D), jnp.float32),
        mesh=mesh,
        scratch_types=[
            pltpu.VMEM((per_w * L,), jnp.int32),
            pltpu.VMEM((crows, D), jnp.float32),
            pltpu.SemaphoreType.DMA,
        ],
        compiler_params=pltpu.CompilerParams(use_tc_tiling_on_sc=False),
    )
    def gather_kernel(tab_hbm, idx_hbm, out_hbm, idx_v, buf0, g0):
        wid = lax.axis_index("s") * _NC + lax.axis_index("c")
        out0 = wid * (per_w * L)
        pltpu.sync_copy(idx_hbm.at[pl.ds(wid * per_w * L, per_w * L)], idx_v)

        def body(c, carry):
            cps = []
            for j in range(JC):
                cps.append(pltpu.async_copy(
                    tab_hbm.at[idx_v.at[pl.ds((c * JC + j) * L, L)]],
                    buf0.at[pl.ds(j * L, L)], g0))
            for cp in cps:
                cp.wait()
            pltpu.sync_copy(buf0, out_hbm.at[pl.ds(out0 + c * crows, crows)])
            return carry

        lax.fori_loop(0, nchunks, body, 0)

    return gather_kernel(tab, idx2d)


def _tower(cts, fm, w1, b1, w2, b2, w3, b3):
    h = None
    for ct, ect in enumerate(cts):
        part = jnp.dot(ect, w1[ct * 128:(ct + 1) * 128, :],
                       preferred_element_type=jnp.float32)
        h = part if h is None else h + part
    h = jnp.maximum(h + b1[None, :], 0.0)
    h = jnp.maximum(
        jnp.dot(h, w2, preferred_element_type=jnp.float32) + b2[None, :], 0.0)
    deep = jnp.dot(h, w3, preferred_element_type=jnp.float32) + b3[None, :]
    z = deep + fm
    return 1.0 / (1.0 + jnp.exp(-z))


def _tc_towers(emb4, smat, params, block_b):
    nb8, nct, _, _ = emb4.shape
    Bn = nb8 * 8
    bbb = block_b // 8

    def body(emb_ref, smat_ref,
             cw1, cb1, cw2, cb2, cw3, cb3,
             tw1, tb1, tw2, tb2, tw3, tb3, out_ref):
        cts = []
        s = None
        sq = None
        for ct in range(nct):
            ect = jnp.reshape(emb_ref[:, ct, :, :], (block_b, 128))
            cts.append(ect)
            spart = jnp.dot(ect, smat_ref[ct * 128:(ct + 1) * 128, :],
                            preferred_element_type=jnp.float32)
            s = spart if s is None else s + spart
            if ct == nct - 1:
                lane = lax.broadcasted_iota(jnp.int32, (1, 128), 1)
                em = jnp.where(lane < 32, ect, 0.0)
            else:
                em = ect
            sqp = jnp.sum(em * em, axis=1, keepdims=True)
            sq = sqp if sq is None else sq + sqp
        ss = jnp.sum(s * s, axis=1, keepdims=True)
        fm = 0.5 * (ss - sq)
        cvr = _tower(cts, fm, cw1[...], cb1[...], cw2[...], cb2[...],
                     cw3[...], cb3[...])
        ctr = _tower(cts, fm, tw1[...], tb1[...], tw2[...], tb2[...],
                     tw3[...], tb3[...])
        res = jnp.concatenate([cvr, ctr, cvr * ctr], axis=1)
        out_ref[...] = jnp.clip(res, 1e-15, 1.0 - 1e-15)

    full = lambda shape: pl.BlockSpec(shape, lambda i: (0,) * len(shape))
    in_specs = [pl.BlockSpec((bbb, nct, 8, 128), lambda i: (i, 0, 0, 0)),
                full(smat.shape)]
    in_specs += [full(p.shape) for p in params]

    return pl.pallas_call(
        body,
        grid=(Bn // block_b,),
        in_specs=in_specs,
        out_specs=pl.BlockSpec((block_b, 3), lambda i: (i, 0)),
        out_shape=jax.ShapeDtypeStruct((Bn, 3), jnp.float32),
    )(emb4, smat, *params)


def kernel(x, tables, cvr_w1, cvr_b1, cvr_w2, cvr_b2, cvr_w3, cvr_b3,
           ctr_w1, ctr_b1, ctr_w2, ctr_b2, ctr_w3, ctr_b3):
    F, V, D = tables.shape
    B = x.shape[0]
    FP = 32
    nct = FP * D // 128

    idx = x + (jnp.arange(F, dtype=jnp.int32) * V)[None, :]
    idx = jnp.concatenate([idx, idx[:, :FP - F]], axis=1)
    idx = idx.reshape(B // 8, 8, nct, 128 // D).transpose(0, 2, 1, 3)
    idx2d = idx.reshape(-1)

    tab = tables.reshape(F * V, D)
    emb = _sc_gather(tab, idx2d)
    emb4 = emb.reshape(B // 8, nct, 8, 128)

    din = FP * D
    r = jnp.arange(din, dtype=jnp.int32)
    smat = jnp.where((r[:, None] % D == jnp.arange(D, dtype=jnp.int32)[None, :])
                     & (r[:, None] < F * D), 1.0, 0.0).astype(jnp.float32)
    pad = ((0, din - F * D), (0, 0))
    params = (jnp.pad(cvr_w1, pad), cvr_b1, cvr_w2, cvr_b2, cvr_w3, cvr_b3,
              jnp.pad(ctr_w1, pad), ctr_b1, ctr_w2, ctr_b2, ctr_w3, ctr_b3)
    return _tc_towers(emb4, smat, params, block_b=512)


# speedup vs baseline: 2.5423x; 2.5423x over previous
import functools

import jax
import jax.numpy as jnp
from jax import lax
from jax.experimental import pallas as pl
from jax.experimental.pallas import tpu as pltpu
from jax.experimental.pallas import tpu_sc as plsc

_NC = 2
_NS = 16


def _sc_gather_t(tabT3, xTflat, B):
    F, D, VR, VC = tabT3.shape
    V = VR * VC
    L = 128
    half = 64
    hb = half * L
    nb2 = B // hb
    R = F * D
    nw = _NC * _NS
    per_w = R // nw
    assert per_w * nw == R and nb2 * hb == B

    mesh = plsc.VectorSubcoreMesh(core_axis_name="c", subcore_axis_name="s")

    @functools.partial(
        pl.kernel,
        out_type=jax.ShapeDtypeStruct((R // 8, B // L, 8, L), jnp.float32),
        mesh=mesh,
        scratch_types=[
            pltpu.VMEM((VR, VC), jnp.float32),
            pltpu.VMEM((hb,), jnp.int32),
            pltpu.VMEM((half, L), jnp.float32),
            pltpu.SemaphoreType.DMA,
        ],
        compiler_params=pltpu.CompilerParams(
            use_tc_tiling_on_sc=False, needs_layout_passes=False),
    )
    def gk(tab_hbm, x_hbm, out_hbm, row_v2, idx_v, out_v, sem):
        wid = lax.axis_index("s") * _NC + lax.axis_index("c")

        def row_body(k, carry):
            r = wid * per_w + k
            f = r // D
            d = lax.rem(r, D)
            rt = r // 8
            rs = lax.rem(r, 8)
            pltpu.sync_copy(tab_hbm.at[f, d], row_v2)

            def half_body(h, carry2):
                pltpu.sync_copy(x_hbm.at[pl.ds(f * B + h * hb, hb)], idx_v)

                def gather16(m, carry3):
                    for l in range(8):
                        vv = idx_v[pl.ds(m * L + l * 16, 16)]
                        vals = plsc.load_gather(
                            row_v2, [vv >> 4, vv & 15])
                        out_v[m, pl.ds(l * 16, 16)] = vals
                    return carry3

                lax.fori_loop(0, half, gather16, 0)
                pltpu.sync_copy(
                    out_v, out_hbm.at[rt, pl.ds(h * half, half), rs, :])
                return carry2

            lax.fori_loop(0, nb2, half_body, 0)
            return carry

        lax.fori_loop(0, per_w, row_body, 0)

    return gk(tabT3, xTflat)


def _tc_towers_t(emb4, smat, params, cbb):
    nrt, nct, _, L = emb4.shape
    Bn = nct * L
    grid = nct // cbb

    def body(emb_ref, smat_ref,
             cw1, cb1, cw2, cb2, cw3, cb3,
             tw1, tb1, tw2, tb2, tw3, tb3, out_ref):
        parts = []
        for j in range(cbb):
            parts.append(jnp.reshape(emb_ref[:, j, :, :], (nrt * 8, L)))
        e = jnp.concatenate(parts, axis=1) if cbb > 1 else parts[0]
        cn = (((0,), (0,)), ((), ()))
        s = lax.dot_general(smat_ref[...], e, cn,
                            preferred_element_type=jnp.float32)
        ss = jnp.sum(s * s, axis=0, keepdims=True)
        sq = jnp.sum(e * e, axis=0, keepdims=True)
        fm = 0.5 * (ss - sq)
        outs = []
        for (w1, b1, w2, b2, w3, b3) in (
                (cw1, cb1, cw2, cb2, cw3, cb3),
                (tw1, tb1, tw2, tb2, tw3, tb3)):
            h = jnp.maximum(lax.dot_general(
                w1[...], e, cn, preferred_element_type=jnp.float32)
                + b1[...], 0.0)
            h = jnp.maximum(lax.dot_general(
                w2[...], h, cn, preferred_element_type=jnp.float32)
                + b2[...], 0.0)
            deep = lax.dot_general(
                w3[...], h, cn, preferred_element_type=jnp.float32) + b3[...]
            z = deep + fm
            outs.append(1.0 / (1.0 + jnp.exp(-z)))
        cvr, ctr = outs
        res = jnp.concatenate([cvr, ctr, cvr * ctr], axis=0)
        res = jnp.clip(res, 1e-15, 1.0 - 1e-15)
        out_ref[...] = jnp.transpose(res, (1, 0))

    full = lambda shape: pl.BlockSpec(shape, lambda i: (0,) * len(shape))
    in_specs = [pl.BlockSpec((nrt, cbb, 8, L), lambda i: (0, i, 0, 0)),
                full(smat.shape)]
    in_specs += [full(p.shape) for p in params]

    return pl.pallas_call(
        body,
        grid=(grid,),
        in_specs=in_specs,
        out_specs=pl.BlockSpec((cbb * L, 3), lambda i: (i, 0)),
        out_shape=jax.ShapeDtypeStruct((Bn, 3), jnp.float32),
    )(emb4, smat, *params)


def kernel(x, tables, cvr_w1, cvr_b1, cvr_w2, cvr_b2, cvr_w3, cvr_b3,
           ctr_w1, ctr_b1, ctr_w2, ctr_b2, ctr_w3, ctr_b3):
    F, V, D = tables.shape
    B = x.shape[0]
    tabT3 = jnp.transpose(tables, (0, 2, 1)).reshape(F, D, V // 16, 16)
    xTflat = jnp.transpose(x, (1, 0)).reshape(-1)
    emb4 = _sc_gather_t(tabT3, xTflat, B)

    din = F * D
    smat = (jnp.arange(din, dtype=jnp.int32)[:, None] % D
            == jnp.arange(D, dtype=jnp.int32)[None, :]).astype(jnp.float32)
    col = lambda b: b[:, None]
    params = (cvr_w1, col(cvr_b1), cvr_w2, col(cvr_b2), cvr_w3, col(cvr_b3),
              ctr_w1, col(ctr_b1), ctr_w2, col(ctr_b2), ctr_w3, col(ctr_b3))
    return _tc_towers_t(emb4, smat, params, cbb=4)


# speedup vs baseline: 5.0332x; 1.9798x over previous
import functools

import jax
import jax.numpy as jnp
from jax import lax
from jax.experimental import pallas as pl
from jax.experimental.pallas import tpu as pltpu
from jax.experimental.pallas import tpu_sc as plsc

_NC = 2
_NS = 16


def _sc_gather_t(tabT3, xTflat, B):
    F, D, V = tabT3.shape
    L = 128
    half = 64
    hb = half * L
    nb2 = B // hb
    R = F * D
    nw = _NC * _NS
    per_w = R // nw
    assert per_w * nw == R and nb2 * hb == B

    mesh = plsc.VectorSubcoreMesh(core_axis_name="c", subcore_axis_name="s")

    @functools.partial(
        pl.kernel,
        out_type=jax.ShapeDtypeStruct((R // 8, B // L, 8, L), jnp.float32),
        mesh=mesh,
        scratch_types=[
            pltpu.VMEM((V,), jnp.float32),
            pltpu.VMEM((hb,), jnp.int32),
            pltpu.VMEM((half, L), jnp.float32),
            pltpu.SemaphoreType.DMA,
        ],
        compiler_params=pltpu.CompilerParams(
            needs_layout_passes=False),
    )
    def gk(tab_hbm, x_hbm, out_hbm, row_v, idx_v, out_v, sem):
        wid = lax.axis_index("s") * _NC + lax.axis_index("c")

        def row_body(k, carry):
            r = wid * per_w + k
            f = r // D
            d = lax.rem(r, D)
            rt = r // 8
            rs = lax.rem(r, 8)
            pltpu.sync_copy(tab_hbm.at[f, d, :], row_v)

            def half_body(h, carry2):
                pltpu.sync_copy(x_hbm.at[pl.ds(f * B + h * hb, hb)], idx_v)

                def gather16(m, carry3):
                    for l in range(8):
                        vv = idx_v[pl.ds(m * L + l * 16, 16)]
                        vals = plsc.load_gather(row_v, [vv])
                        out_v[m, pl.ds(l * 16, 16)] = vals
                    return carry3

                lax.fori_loop(0, half, gather16, 0)
                pltpu.sync_copy(
                    out_v, out_hbm.at[rt, pl.ds(h * half, half), rs, :])
                return carry2

            lax.fori_loop(0, nb2, half_body, 0)
            return carry

        lax.fori_loop(0, per_w, row_body, 0)

    return gk(tabT3, xTflat)


def _tc_towers_t(emb4, smat, params, cbb):
    nrt, nct, _, L = emb4.shape
    Bn = nct * L
    grid = nct // cbb

    def body(emb_ref, smat_ref,
             cw1, cb1, cw2, cb2, cw3, cb3,
             tw1, tb1, tw2, tb2, tw3, tb3, out_ref):
        parts = []
        for j in range(cbb):
            parts.append(jnp.reshape(emb_ref[:, j, :, :], (nrt * 8, L)))
        e = jnp.concatenate(parts, axis=1) if cbb > 1 else parts[0]
        cn = (((0,), (0,)), ((), ()))
        s = lax.dot_general(smat_ref[...], e, cn,
                            preferred_element_type=jnp.float32)
        ss = jnp.sum(s * s, axis=0, keepdims=True)
        sq = jnp.sum(e * e, axis=0, keepdims=True)
        fm = 0.5 * (ss - sq)
        outs = []
        for (w1, b1, w2, b2, w3, b3) in (
                (cw1, cb1, cw2, cb2, cw3, cb3),
                (tw1, tb1, tw2, tb2, tw3, tb3)):
            h = jnp.maximum(lax.dot_general(
                w1[...], e, cn, preferred_element_type=jnp.float32)
                + b1[...], 0.0)
            h = jnp.maximum(lax.dot_general(
                w2[...], h, cn, preferred_element_type=jnp.float32)
                + b2[...], 0.0)
            deep = lax.dot_general(
                w3[...], h, cn, preferred_element_type=jnp.float32) + b3[...]
            z = deep + fm
            outs.append(1.0 / (1.0 + jnp.exp(-z)))
        cvr, ctr = outs
        res = jnp.concatenate([cvr, ctr, cvr * ctr], axis=0)
        res = jnp.clip(res, 1e-15, 1.0 - 1e-15)
        out_ref[...] = jnp.transpose(res, (1, 0))

    full = lambda shape: pl.BlockSpec(shape, lambda i: (0,) * len(shape))
    in_specs = [pl.BlockSpec((nrt, cbb, 8, L), lambda i: (0, i, 0, 0)),
                full(smat.shape)]
    in_specs += [full(p.shape) for p in params]

    return pl.pallas_call(
        body,
        grid=(grid,),
        in_specs=in_specs,
        out_specs=pl.BlockSpec((cbb * L, 3), lambda i: (i, 0)),
        out_shape=jax.ShapeDtypeStruct((Bn, 3), jnp.float32),
    )(emb4, smat, *params)


def kernel(x, tables, cvr_w1, cvr_b1, cvr_w2, cvr_b2, cvr_w3, cvr_b3,
           ctr_w1, ctr_b1, ctr_w2, ctr_b2, ctr_w3, ctr_b3):
    F, V, D = tables.shape
    B = x.shape[0]
    tabT3 = jnp.transpose(tables, (0, 2, 1))
    xTflat = jnp.transpose(x, (1, 0)).reshape(-1)
    emb4 = _sc_gather_t(tabT3, xTflat, B)

    din = F * D
    smat = (jnp.arange(din, dtype=jnp.int32)[:, None] % D
            == jnp.arange(D, dtype=jnp.int32)[None, :]).astype(jnp.float32)
    col = lambda b: b[:, None]
    params = (cvr_w1, col(cvr_b1), cvr_w2, col(cvr_b2), cvr_w3, col(cvr_b3),
              ctr_w1, col(ctr_b1), ctr_w2, col(ctr_b2), ctr_w3, col(ctr_b3))
    return _tc_towers_t(emb4, smat, params, cbb=4)
